# baseline (speedup 1.0000x reference)
import functools
import jax
import jax.numpy as jnp
from jax.experimental import pallas as pl
from jax.experimental.pallas import tpu as pltpu

T = 2048
D = 1024
H = 16
KVH = 4
HD = 64
E = 8
TOPK = 2
I = 768
THETA = 1000000.0
EPS = 1e-6

BT = 256
BK = 256
EPAD = 128
NEG = jnp.finfo(jnp.float32).min


def _pre_kernel(pos_ref, x_ref, w_ref, ln1_ref, qn_ref, kn_ref, q_ref, k_ref, v_ref):
    x = x_ref[...]
    h = x * jax.lax.rsqrt(jnp.mean(x * x, axis=-1, keepdims=True) + EPS) * ln1_ref[...]
    qkv = jnp.dot(h, w_ref[...], preferred_element_type=jnp.float32)
    pos = pos_ref[...].astype(jnp.float32)
    half_iota = jax.lax.broadcasted_iota(jnp.int32, (1, HD // 2), 1).astype(jnp.float32)
    inv_freq = jnp.exp(half_iota * (-2.0 / HD) * jnp.log(THETA))
    freqs = pos * inv_freq
    cs = jnp.cos(freqs)
    sn = jnp.sin(freqs)
    qn = qn_ref[...]
    kn = kn_ref[...]
    hf = HD // 2
    for hh in range(H):
        qh = qkv[:, hh * HD:(hh + 1) * HD]
        qh = qh * jax.lax.rsqrt(jnp.mean(qh * qh, axis=-1, keepdims=True) + EPS) * qn
        x1 = qh[:, :hf]
        x2 = qh[:, hf:]
        q_ref[hh] = jnp.concatenate([x1 * cs - x2 * sn, x2 * cs + x1 * sn], axis=1)
    for hh in range(KVH):
        kh = qkv[:, H * HD + hh * HD:H * HD + (hh + 1) * HD]
        kh = kh * jax.lax.rsqrt(jnp.mean(kh * kh, axis=-1, keepdims=True) + EPS) * kn
        x1 = kh[:, :hf]
        x2 = kh[:, hf:]
        k_ref[hh] = jnp.concatenate([x1 * cs - x2 * sn, x2 * cs + x1 * sn], axis=1)
        v_ref[hh] = qkv[:, (H + KVH) * HD + hh * HD:(H + KVH) * HD + (hh + 1) * HD]


def _flash_kernel(q_ref, k_ref, v_ref, o_ref):
    tq = pl.program_id(1)
    q = q_ref[0] * (HD ** -0.5)

    def body(j, carry):
        m, l, acc = carry
        k = k_ref[0, pl.ds(j * BK, BK), :]
        s = jax.lax.dot_general(q, k, (((1,), (1,)), ((), ())),
                                preferred_element_type=jnp.float32)
        rows = tq * BT + jax.lax.broadcasted_iota(jnp.int32, (BT, BK), 0)
        cols = j * BK + jax.lax.broadcasted_iota(jnp.int32, (BT, BK), 1)
        s = jnp.where(rows >= cols, s, NEG)
        m_new = jnp.maximum(m, jnp.max(s, axis=-1, keepdims=True))
        p = jnp.exp(s - m_new)
        alpha = jnp.exp(m - m_new)
        l_new = l * alpha + jnp.sum(p, axis=-1, keepdims=True)
        v = v_ref[0, pl.ds(j * BK, BK), :]
        acc_new = acc * alpha + jnp.dot(p, v, preferred_element_type=jnp.float32)
        return m_new, l_new, acc_new

    m0 = jnp.full((BT, 1), NEG, jnp.float32)
    l0 = jnp.zeros((BT, 1), jnp.float32)
    a0 = jnp.zeros((BT, HD), jnp.float32)
    m, l, acc = jax.lax.fori_loop(0, tq + 1, body, (m0, l0, a0))
    o_ref[0] = acc / l


def _post_kernel(o_ref, res_ref, wo_ref, ln2_ref, gate_ref, h1_ref, h2_ref, w_ref):
    attn = jnp.zeros((BT, D), jnp.float32)
    for hh in range(H):
        attn = attn + jnp.dot(o_ref[hh], wo_ref[pl.ds(hh * HD, HD), :],
                              preferred_element_type=jnp.float32)
    h1 = res_ref[...] + attn
    h1_ref[...] = h1
    h2 = h1 * jax.lax.rsqrt(jnp.mean(h1 * h1, axis=-1, keepdims=True) + EPS) * ln2_ref[...]
    h2_ref[...] = h2
    logits = jnp.dot(h2, gate_ref[...], preferred_element_type=jnp.float32)
    col = jax.lax.broadcasted_iota(jnp.int32, (BT, EPAD), 1)
    valid = col < E
    lm = jnp.where(valid, logits, NEG)
    mx = jnp.max(lm, axis=-1, keepdims=True)
    p = jnp.where(valid, jnp.exp(lm - mx), 0.0)
    rw = p / jnp.sum(p, axis=-1, keepdims=True)
    m1 = jnp.max(rw, axis=-1, keepdims=True)
    i1 = jnp.min(jnp.where(rw == m1, col, EPAD), axis=-1, keepdims=True)
    f1 = col == i1
    rw2 = jnp.where(f1, -1.0, rw)
    m2 = jnp.max(rw2, axis=-1, keepdims=True)
    i2 = jnp.min(jnp.where(rw2 == m2, col, EPAD), axis=-1, keepdims=True)
    f2 = col == i2
    denom = m1 + m2
    w = (jnp.where(f1, m1, 0.0) + jnp.where(f2, m2, 0.0)) / denom
    w_ref[...] = w


def _moe_dense_kernel(h2_ref, wgu_ref, wd_ref, w_ref, h1_ref, out_ref):
    e = pl.program_id(1)

    @pl.when(e == 0)
    def _init():
        out_ref[...] = h1_ref[...]

    x = h2_ref[...]
    gu = jnp.dot(x, wgu_ref[0], preferred_element_type=jnp.float32)
    g = gu[:, :I]
    u = gu[:, I:]
    act = (g / (1.0 + jnp.exp(-g))) * u
    dn = jnp.dot(act, wd_ref[0], preferred_element_type=jnp.float32)
    col = jax.lax.broadcasted_iota(jnp.int32, (BT, EPAD), 1)
    wcol = jnp.sum(jnp.where(col == e, w_ref[...], 0.0), axis=-1, keepdims=True)
    out_ref[...] += wcol * dn


def kernel(hidden_states, positions, W_qkv, q_norm_w, k_norm_w, W_o, ln1_w, ln2_w,
           gate_w, W_gate_up, W_down):
    pos2 = positions.reshape(T, 1)
    ln1 = ln1_w.reshape(1, D)
    ln2 = ln2_w.reshape(1, D)
    qn = q_norm_w.reshape(1, HD)
    kn = k_norm_w.reshape(1, HD)
    gate_pad = jnp.concatenate([gate_w, jnp.zeros((D, EPAD - E), jnp.float32)], axis=1)

    nt = T // BT
    q, k, v = pl.pallas_call(
        _pre_kernel,
        grid=(nt,),
        in_specs=[
            pl.BlockSpec((BT, 1), lambda t: (t, 0)),
            pl.BlockSpec((BT, D), lambda t: (t, 0)),
            pl.BlockSpec((D, (H + 2 * KVH) * HD), lambda t: (0, 0)),
            pl.BlockSpec((1, D), lambda t: (0, 0)),
            pl.BlockSpec((1, HD), lambda t: (0, 0)),
            pl.BlockSpec((1, HD), lambda t: (0, 0)),
        ],
        out_specs=[
            pl.BlockSpec((H, BT, HD), lambda t: (0, t, 0)),
            pl.BlockSpec((KVH, BT, HD), lambda t: (0, t, 0)),
            pl.BlockSpec((KVH, BT, HD), lambda t: (0, t, 0)),
        ],
        out_shape=[
            jax.ShapeDtypeStruct((H, T, HD), jnp.float32),
            jax.ShapeDtypeStruct((KVH, T, HD), jnp.float32),
            jax.ShapeDtypeStruct((KVH, T, HD), jnp.float32),
        ],
    )(pos2, hidden_states, W_qkv, ln1, qn, kn)

    rep = H // KVH
    o = pl.pallas_call(
        _flash_kernel,
        grid=(H, nt),
        in_specs=[
            pl.BlockSpec((1, BT, HD), lambda h, t: (h, t, 0)),
            pl.BlockSpec((1, T, HD), lambda h, t: (h // rep, 0, 0)),
            pl.BlockSpec((1, T, HD), lambda h, t: (h // rep, 0, 0)),
        ],
        out_specs=pl.BlockSpec((1, BT, HD), lambda h, t: (h, t, 0)),
        out_shape=jax.ShapeDtypeStruct((H, T, HD), jnp.float32),
    )(q, k, v)

    h1, h2, w = pl.pallas_call(
        _post_kernel,
        grid=(nt,),
        in_specs=[
            pl.BlockSpec((H, BT, HD), lambda t: (0, t, 0)),
            pl.BlockSpec((BT, D), lambda t: (t, 0)),
            pl.BlockSpec((H * HD, D), lambda t: (0, 0)),
            pl.BlockSpec((1, D), lambda t: (0, 0)),
            pl.BlockSpec((D, EPAD), lambda t: (0, 0)),
        ],
        out_specs=[
            pl.BlockSpec((BT, D), lambda t: (t, 0)),
            pl.BlockSpec((BT, D), lambda t: (t, 0)),
            pl.BlockSpec((BT, EPAD), lambda t: (t, 0)),
        ],
        out_shape=[
            jax.ShapeDtypeStruct((T, D), jnp.float32),
            jax.ShapeDtypeStruct((T, D), jnp.float32),
            jax.ShapeDtypeStruct((T, EPAD), jnp.float32),
        ],
    )(o, hidden_states, W_o, ln2, gate_pad)

    out = pl.pallas_call(
        _moe_dense_kernel,
        grid=(nt, E),
        in_specs=[
            pl.BlockSpec((BT, D), lambda t, e: (t, 0)),
            pl.BlockSpec((1, D, 2 * I), lambda t, e: (e, 0, 0)),
            pl.BlockSpec((1, I, D), lambda t, e: (e, 0, 0)),
            pl.BlockSpec((BT, EPAD), lambda t, e: (t, 0)),
            pl.BlockSpec((BT, D), lambda t, e: (t, 0)),
        ],
        out_specs=pl.BlockSpec((BT, D), lambda t, e: (t, 0)),
        out_shape=jax.ShapeDtypeStruct((T, D), jnp.float32),
    )(h2, W_gate_up, W_down, w, h1)

    return out
